# baseline (speedup 1.0000x reference)
import functools

import jax
import jax.numpy as jnp
from jax.experimental import pallas as pl


def _uw_body(u3t_ref, u2f_ref, u1_ref, w3_ref, w2_ref, w1_ref,
             uw3_ref, uw2_ref, uw1_ref):
    E = w3_ref.shape[0]
    for e in range(E):
        uw3_ref[e] = jnp.dot(u3t_ref[...], w3_ref[e],
                             preferred_element_type=jnp.float32)
        uw2_ref[e] = jnp.dot(u2f_ref[...], w2_ref[e],
                             preferred_element_type=jnp.float32)
        uw1_ref[e] = jnp.dot(u1_ref[...], w1_ref[e],
                             preferred_element_type=jnp.float32)


def _main_body(types_ref, x_ref, uw3_ref, uw2_ref, uw1_ref, o_ref, *, ba, nl):
    for a in range(ba):
        e = types_ref[a, 0]
        xa = x_ref[a]
        acc = uw2_ref[e]
        for k in range(nl):
            acc = acc + uw3_ref[e, k] * xa[k][None, :]
        c2 = acc.reshape(nl, nl, xa.shape[-1])
        c1 = jnp.sum(c2 * xa[None], axis=1) + uw1_ref[e]
        o_ref[a] = jnp.sum(c1 * xa, axis=0)


def kernel(x, atom_types, U3, U2, U1, W3, W2, W1):
    A, nl, C = x.shape
    E, nw3, _ = W3.shape
    nw2 = W2.shape[1]
    nij = nl * nl

    u3t = U3.transpose(2, 0, 1, 3).reshape(nl * nij, nw3)
    u2f = U2.reshape(nij, nw2)

    uw3, uw2, uw1 = pl.pallas_call(
        _uw_body,
        out_shape=(
            jax.ShapeDtypeStruct((E, nl * nij, C), jnp.float32),
            jax.ShapeDtypeStruct((E, nij, C), jnp.float32),
            jax.ShapeDtypeStruct((E, nl, C), jnp.float32),
        ),
    )(u3t, u2f, U1, W3, W2, W1)
    uw3 = uw3.reshape(E, nl, nij, C)

    BA = 8
    types2d = atom_types.reshape(A, 1).astype(jnp.int32)
    out = pl.pallas_call(
        functools.partial(_main_body, ba=BA, nl=nl),
        grid=(A // BA,),
        in_specs=[
            pl.BlockSpec((BA, 1), lambda b: (b, 0)),
            pl.BlockSpec((BA, nl, C), lambda b: (b, 0, 0)),
            pl.BlockSpec((E, nl, nij, C), lambda b: (0, 0, 0, 0)),
            pl.BlockSpec((E, nij, C), lambda b: (0, 0, 0)),
            pl.BlockSpec((E, nl, C), lambda b: (0, 0, 0)),
        ],
        out_specs=pl.BlockSpec((BA, C), lambda b: (b, 0)),
        out_shape=jax.ShapeDtypeStruct((A, C), jnp.float32),
    )(types2d, x, uw3, uw2, uw1)
    return out


# speedup vs baseline: 1.2174x; 1.2174x over previous
import functools
import itertools

import numpy as np

import jax
import jax.numpy as jnp
from jax.experimental import pallas as pl


def _pair_index(nl):
    i_idx, j_idx = [], []
    for j in range(nl):
        for i in range(j + 1):
            i_idx.append(i)
            j_idx.append(j)
    return np.array(i_idx, np.int32), np.array(j_idx, np.int32)


def _uw_body(g_ref, scale_ref, u2s_ref, u1_ref, w3_ref, w2_ref, w1_ref,
             uw3_ref, uw2_ref, uw1_ref):
    E = w3_ref.shape[0]
    u3tm = g_ref[0]
    for p in range(1, g_ref.shape[0]):
        u3tm = u3tm + g_ref[p]
    u3tm = u3tm * scale_ref[...]
    for e in range(E):
        uw3_ref[e] = jax.lax.dot_general(
            u3tm, w3_ref[e], (((1,), (0,)), ((), ())),
            preferred_element_type=jnp.float32)
        uw2_ref[e] = jax.lax.dot_general(
            u2s_ref[...], w2_ref[e], (((1,), (0,)), ((), ())),
            preferred_element_type=jnp.float32)
        uw1_ref[e] = jax.lax.dot_general(
            u1_ref[...], w1_ref[e], (((1,), (0,)), ((), ())),
            preferred_element_type=jnp.float32)


def _main_body(types_ref, x_ref, uw3_ref, uw2_ref, uw1_ref, pi_ref, pj_ref,
               o_ref, *, ba, nl, npair):
    ones = jnp.ones((1, npair + nl), jnp.float32)
    for a in range(ba):
        e = types_ref[a, 0]
        xa = x_ref[a]
        xi = jax.lax.dot_general(pi_ref[...], xa, (((1,), (0,)), ((), ())),
                                 preferred_element_type=jnp.float32)
        xj = jax.lax.dot_general(pj_ref[...], xa, (((1,), (0,)), ((), ())),
                                 preferred_element_type=jnp.float32)
        acc = uw2_ref[e]
        for k in range(nl):
            t8 = (-((k + 1) * (k + 2) // 2) // 8) * -8
            upd = acc[:t8] + uw3_ref[e, k, :t8] * xa[k][None, :]
            acc = jnp.concatenate([upd, acc[t8:]], axis=0) if t8 < npair else upd
        prod = jnp.concatenate([acc * xi * xj, uw1_ref[e] * xa], axis=0)
        o_ref[a] = jax.lax.dot_general(
            ones, prod, (((1,), (0,)), ((), ())),
            preferred_element_type=jnp.float32)[0]


def kernel(x, atom_types, U3, U2, U1, W3, W2, W1):
    A, nl, C = x.shape
    E, nw3, _ = W3.shape
    nw2 = W2.shape[1]
    i_idx, j_idx = _pair_index(nl)
    npair = len(i_idx)

    gs = []
    for p in itertools.permutations((0, 1, 2)):
        up = jnp.transpose(U3, p + (3,))
        gs.append(jnp.transpose(up[i_idx, j_idx], (1, 0, 2)))
    g = jnp.stack(gs).reshape(6, nl * npair, nw3)

    knum = np.arange(nl)[:, None]
    mult = np.where((i_idx[None] == j_idx[None]) & (j_idx[None] == knum), 6.0,
                    np.where((i_idx[None] == j_idx[None])
                             | (j_idx[None] == knum), 2.0, 1.0))
    scale = np.where(j_idx[None] <= knum, 1.0 / mult, 0.0)
    scale = jnp.asarray(scale.reshape(nl * npair, 1), jnp.float32)

    u2s = (U2[i_idx, j_idx] + U2[j_idx, i_idx]) \
        * jnp.asarray(np.where(i_idx == j_idx, 0.5, 1.0)[:, None], jnp.float32)

    uw3, uw2, uw1 = pl.pallas_call(
        _uw_body,
        out_shape=(
            jax.ShapeDtypeStruct((E, nl * npair, C), jnp.float32),
            jax.ShapeDtypeStruct((E, npair, C), jnp.float32),
            jax.ShapeDtypeStruct((E, nl, C), jnp.float32),
        ),
    )(g, scale, u2s, U1, W3, W2, W1)
    uw3 = uw3.reshape(E, nl, npair, C)

    p_i = jnp.asarray(np.eye(nl, dtype=np.float32)[i_idx])
    p_j = jnp.asarray(np.eye(nl, dtype=np.float32)[j_idx])

    BA = 8
    types2d = atom_types.reshape(A, 1).astype(jnp.int32)
    out = pl.pallas_call(
        functools.partial(_main_body, ba=BA, nl=nl, npair=npair),
        grid=(A // BA,),
        in_specs=[
            pl.BlockSpec((BA, 1), lambda b: (b, 0)),
            pl.BlockSpec((BA, nl, C), lambda b: (b, 0, 0)),
            pl.BlockSpec((E, nl, npair, C), lambda b: (0, 0, 0, 0)),
            pl.BlockSpec((E, npair, C), lambda b: (0, 0, 0)),
            pl.BlockSpec((E, nl, C), lambda b: (0, 0, 0)),
            pl.BlockSpec((npair, nl), lambda b: (0, 0)),
            pl.BlockSpec((npair, nl), lambda b: (0, 0)),
        ],
        out_specs=pl.BlockSpec((BA, C), lambda b: (b, 0)),
        out_shape=jax.ShapeDtypeStruct((A, C), jnp.float32),
    )(types2d, x, uw3, uw2, uw1, p_i, p_j)
    return out
